# baseline (device time: 49839 ns/iter reference)
import jax
import jax.numpy as jnp
from jax import lax
from jax.experimental import pallas as pl
from jax.experimental.pallas import tpu as pltpu

N_DEV = 4
CHUNK = 512


def _scan8(y):
    row = lax.broadcasted_iota(jnp.int32, y.shape, 1)
    for s in (1, 2, 4):
        shifted = pltpu.roll(y, s, 1)
        y = y * jnp.where(row < s, jnp.float32(1.0), shifted)
    return y


def _exclusive8(y):
    row = lax.broadcasted_iota(jnp.int32, y.shape, 1)
    return jnp.where(row < 1, jnp.float32(1.0), pltpu.roll(y, 1, 1))


def _cumprod512(xc):
    n = xc.shape[1]
    a0 = _scan8(xc.reshape(CHUNK // 8, 8, n))
    a1 = _scan8(a0[:, 7, :].reshape(CHUNK // 64, 8, n))
    a2 = _scan8(a1[:, 7, :].reshape(1, 8, n))
    p1 = _exclusive8(a2).reshape(CHUNK // 64, n)
    p0 = (_exclusive8(a1) * p1.reshape(CHUNK // 64, 1, n)).reshape(CHUNK // 8, n)
    y = a0 * p0.reshape(CHUNK // 8, 1, n)
    return y.reshape(CHUNK, n), a2[:, 7, :]


def kernel(x):
    m, n = x.shape
    n_chunks = m // CHUNK

    def body(x_hbm, out_hbm, s_buf, in_buf, totals_ref, in_sems, out_sems,
             send_sems, recv_sems):
        my = lax.axis_index("i")

        barrier_sem = pltpu.get_barrier_semaphore()
        for off in range(1, N_DEV):
            pl.semaphore_signal(
                barrier_sem,
                inc=1,
                device_id=((my + off) % N_DEV,),
                device_id_type=pl.DeviceIdType.MESH,
            )
        pl.semaphore_wait(barrier_sem, N_DEV - 1)

        def in_copy(k):
            return pltpu.make_async_copy(
                x_hbm.at[pl.ds(k * CHUNK, CHUNK)],
                in_buf.at[k % 2],
                in_sems.at[k % 2],
            )

        in_copy(0).start()
        carry = jnp.ones((1, n), jnp.float32)
        for k in range(n_chunks):
            if k + 1 < n_chunks:
                in_copy(k + 1).start()
            in_copy(k).wait()
            yc, tc = _cumprod512(in_buf[k % 2])
            s_buf[k] = yc * carry
            carry = carry * tc

        totals_ref[pl.ds(my, 1), :] = carry
        sends = []
        for k in range(N_DEV - 1):
            rdma = pltpu.make_async_remote_copy(
                src_ref=totals_ref.at[pl.ds(my, 1)],
                dst_ref=totals_ref.at[pl.ds(my, 1)],
                send_sem=send_sems.at[k],
                recv_sem=recv_sems.at[k],
                device_id=((my + k + 1) % N_DEV,),
                device_id_type=pl.DeviceIdType.MESH,
            )
            rdma.start()
            sends.append(rdma)
        for k in range(N_DEV - 1):
            src_row = (my - 1 - k) % N_DEV
            recv = pltpu.make_async_remote_copy(
                src_ref=totals_ref.at[pl.ds(src_row, 1)],
                dst_ref=totals_ref.at[pl.ds(src_row, 1)],
                send_sem=send_sems.at[k],
                recv_sem=recv_sems.at[k],
                device_id=(my,),
                device_id_type=pl.DeviceIdType.MESH,
            )
            recv.wait_recv()
        for rdma in sends:
            rdma.wait_send()

        totals = totals_ref[:, :]
        rid = lax.broadcasted_iota(jnp.int32, (N_DEV, n), 0)
        factors = jnp.where(rid < my, totals, jnp.ones_like(totals))
        prefix = factors[0] * factors[1] * factors[2] * factors[3]
        prefix = prefix[None, :]

        out_copies = []
        for k in range(n_chunks):
            s_buf[k] = s_buf[k] * prefix
            cp = pltpu.make_async_copy(
                s_buf.at[k],
                out_hbm.at[pl.ds(k * CHUNK, CHUNK)],
                out_sems.at[k],
            )
            cp.start()
            out_copies.append(cp)
        for cp in out_copies:
            cp.wait()

    return pl.pallas_call(
        body,
        out_shape=jax.ShapeDtypeStruct((m, n), jnp.float32),
        in_specs=[pl.BlockSpec(memory_space=pltpu.MemorySpace.HBM)],
        out_specs=pl.BlockSpec(memory_space=pltpu.MemorySpace.HBM),
        scratch_shapes=[
            pltpu.VMEM((n_chunks, CHUNK, n), jnp.float32),
            pltpu.VMEM((2, CHUNK, n), jnp.float32),
            pltpu.VMEM((N_DEV, n), jnp.float32),
            pltpu.SemaphoreType.DMA((2,)),
            pltpu.SemaphoreType.DMA((n_chunks,)),
            pltpu.SemaphoreType.DMA((N_DEV - 1,)),
            pltpu.SemaphoreType.DMA((N_DEV - 1,)),
        ],
        compiler_params=pltpu.CompilerParams(
            collective_id=0, vmem_limit_bytes=100 * 1024 * 1024
        ),
    )(x)
